# Initial kernel scaffold; baseline (speedup 1.0000x reference)
import jax
import jax.numpy as jnp
from jax.experimental import pallas as pl


def kernel(x, edge_index, edge_attr, weight, w_ih, w_hh, b_ih, b_hh):
    raise NotImplementedError("write your pallas kernel here")


# speedup vs baseline: 5.7541x; 5.7541x over previous
import functools

import jax
import jax.numpy as jnp
from jax import lax
from jax.experimental import pallas as pl
from jax.experimental.pallas import tpu as pltpu
from jax.experimental.pallas import tpu_sc as plsc

N = 10000
C = 256
E = 160000
CH = C // 2
NS = 16
EPT = E // NS
CHUNK = 125
NCHUNK = EPT // CHUNK
N_PAD = 10240
RPT = N_PAD // NS


def _sc_segment_sum(m2, src2, dst3, zeros):
    mesh = plsc.VectorSubcoreMesh(core_axis_name="c", subcore_axis_name="s")

    @functools.partial(
        pl.kernel,
        out_type=jax.ShapeDtypeStruct((2 * N_PAD, CH), jnp.float32),
        mesh=mesh,
        scratch_types=[
            pltpu.VMEM((NCHUNK, CHUNK), jnp.int32),
            pltpu.VMEM((NCHUNK, CHUNK), jnp.int32),
            pltpu.VMEM((CHUNK, CH), jnp.float32),
            pltpu.VMEM_SHARED((N_PAD, CH), jnp.float32),
            pltpu.SemaphoreType.DMA,
        ],
    )
    def k(m2_hbm, src_hbm, dst_hbm, zeros_hbm, out_hbm, src_v, dst_v, buf, acc, sem):
        cid = lax.axis_index("c")
        sid = lax.axis_index("s")
        pltpu.sync_copy(src_hbm.at[cid, sid], src_v)
        pltpu.sync_copy(dst_hbm.at[sid], dst_v)
        pltpu.sync_copy(zeros_hbm.at[pl.ds(sid * RPT, RPT)],
                        acc.at[pl.ds(sid * RPT, RPT)])
        plsc.subcore_barrier()

        def body(j, carry):
            pltpu.async_copy(m2_hbm.at[src_v.at[j]], buf, sem).wait()
            pltpu.sync_copy(buf, acc.at[dst_v.at[j]], add=True)
            return carry

        lax.fori_loop(0, NCHUNK, body, 0)
        plsc.subcore_barrier()
        pltpu.sync_copy(acc.at[pl.ds(sid * RPT, RPT)],
                        out_hbm.at[pl.ds(cid * N_PAD + sid * RPT, RPT)])

    return k(m2, src2, dst3, zeros)


_BN = 1000


def _tc_mm(x, w):
    def body(x_ref, w_ref, out_ref):
        m = jnp.dot(x_ref[...], w_ref[...], preferred_element_type=jnp.float32)
        out_ref[0] = m[:, :CH]
        out_ref[1] = m[:, CH:]

    return pl.pallas_call(
        body,
        grid=(N // _BN,),
        in_specs=[
            pl.BlockSpec((_BN, C), lambda i: (i, 0)),
            pl.BlockSpec((C, C), lambda i: (0, 0)),
        ],
        out_specs=pl.BlockSpec((2, _BN, CH), lambda i: (0, i, 0)),
        out_shape=jax.ShapeDtypeStruct((2, N, CH), jnp.float32),
    )(x, w)


def _gru_block(s_ref, h_ref, wih_ref, whh_ref, bih_ref, bhh_ref):
    agg = jnp.concatenate([s_ref[0], s_ref[1]], axis=1)
    h = h_ref[...]
    gi = lax.dot_general(agg, wih_ref[...], (((1,), (1,)), ((), ())),
                         preferred_element_type=jnp.float32) + bih_ref[...]
    gh = lax.dot_general(h, whh_ref[...], (((1,), (1,)), ((), ())),
                         preferred_element_type=jnp.float32) + bhh_ref[...]
    r = jax.nn.sigmoid(gi[:, :C] + gh[:, :C])
    z = jax.nn.sigmoid(gi[:, C:2 * C] + gh[:, C:2 * C])
    n = jnp.tanh(gi[:, 2 * C:] + r * gh[:, 2 * C:])
    return (1.0 - z) * n + z * h


_SPECS = [
    pl.BlockSpec((2, _BN, CH), lambda i: (0, i, 0)),
    pl.BlockSpec((_BN, C), lambda i: (i, 0)),
    pl.BlockSpec((3 * C, C), lambda i: (0, 0)),
    pl.BlockSpec((3 * C, C), lambda i: (0, 0)),
    pl.BlockSpec((1, 3 * C), lambda i: (0, 0)),
    pl.BlockSpec((1, 3 * C), lambda i: (0, 0)),
]


def _tc_gru_mm(s3, h, w_next, w_ih, w_hh, b_ih2, b_hh2):
    def body(s_ref, h_ref, wih_ref, whh_ref, bih_ref, bhh_ref, wn_ref,
             hout_ref, mout_ref):
        hn = _gru_block(s_ref, h_ref, wih_ref, whh_ref, bih_ref, bhh_ref)
        hout_ref[...] = hn
        m = jnp.dot(hn, wn_ref[...], preferred_element_type=jnp.float32)
        mout_ref[0] = m[:, :CH]
        mout_ref[1] = m[:, CH:]

    return pl.pallas_call(
        body,
        grid=(N // _BN,),
        in_specs=_SPECS + [pl.BlockSpec((C, C), lambda i: (0, 0))],
        out_specs=[
            pl.BlockSpec((_BN, C), lambda i: (i, 0)),
            pl.BlockSpec((2, _BN, CH), lambda i: (0, i, 0)),
        ],
        out_shape=[
            jax.ShapeDtypeStruct((N, C), jnp.float32),
            jax.ShapeDtypeStruct((2, N, CH), jnp.float32),
        ],
    )(s3, h, w_ih, w_hh, b_ih2, b_hh2, w_next)


def _tc_gru(s3, h, w_ih, w_hh, b_ih2, b_hh2):
    def body(s_ref, h_ref, wih_ref, whh_ref, bih_ref, bhh_ref, hout_ref):
        hout_ref[...] = _gru_block(s_ref, h_ref, wih_ref, whh_ref, bih_ref, bhh_ref)

    return pl.pallas_call(
        body,
        grid=(N // _BN,),
        in_specs=_SPECS,
        out_specs=pl.BlockSpec((_BN, C), lambda i: (i, 0)),
        out_shape=jax.ShapeDtypeStruct((N, C), jnp.float32),
    )(s3, h, w_ih, w_hh, b_ih2, b_hh2)


def kernel(x, edge_index, edge_attr, weight, w_ih, w_hh, b_ih, b_hh):
    src = edge_index[0].astype(jnp.int32)
    dst = edge_index[1].astype(jnp.int32)
    srcr = src.reshape(NS, NCHUNK, CHUNK)
    src2 = jnp.stack([srcr, srcr + N])
    dst3 = dst.reshape(NS, NCHUNK, CHUNK)
    zeros = jnp.zeros((N_PAD, CH), jnp.float32)
    b_ih2 = b_ih.reshape(1, 3 * C)
    b_hh2 = b_hh.reshape(1, 3 * C)
    num_layers = weight.shape[0]
    h = x
    m3 = _tc_mm(x, weight[0])
    for i in range(num_layers):
        s2 = _sc_segment_sum(m3.reshape(2 * N, CH), src2, dst3, zeros)
        s3 = s2.reshape(2, N_PAD, CH)
        if i + 1 < num_layers:
            h, m3 = _tc_gru_mm(s3, h, weight[i + 1], w_ih, w_hh, b_ih2, b_hh2)
        else:
            h = _tc_gru(s3, h, w_ih, w_hh, b_ih2, b_hh2)
    return h
